# Initial kernel scaffold; baseline (speedup 1.0000x reference)
import jax
import jax.numpy as jnp
from jax.experimental import pallas as pl


def kernel(x, duration, max_len):
    raise NotImplementedError("write your pallas kernel here")


# speedup vs baseline: 49.2009x; 49.2009x over previous
import functools

import jax
import jax.numpy as jnp
from jax import lax
from jax.experimental import pallas as pl
from jax.experimental.pallas import tpu as pltpu
from jax.experimental.pallas import tpu_sc as plsc

B = 16
T = 512
D = 256
MAXLEN = 2048
L = 16
NC = 2
NS = 16
NW = NC * NS
POS_PER_W = B * MAXLEN // NW
CHUNK = 128
NCHUNKS = POS_PER_W // CHUNK
DMAX = 7


def _build():
    mesh = plsc.VectorSubcoreMesh(core_axis_name="c", subcore_axis_name="s")

    @functools.partial(
        pl.kernel,
        mesh=mesh,
        compiler_params=pltpu.CompilerParams(needs_layout_passes=False),
        out_type=(
            jax.ShapeDtypeStruct((B * MAXLEN, D), jnp.float32),
            jax.ShapeDtypeStruct((B,), jnp.int32),
        ),
        scratch_types=[
            pltpu.VMEM((B, T), jnp.int32),
            pltpu.VMEM((MAXLEN,), jnp.int32),
            pltpu.VMEM((CHUNK, D), jnp.float32),
            pltpu.VMEM((L,), jnp.int32),
            pltpu.SemaphoreType.DMA,
        ],
    )
    def expand(x_hbm, dur_hbm, out_hbm, mel_hbm, dur_all, idx_v, rows_v,
               mel_v, sem):
        cid = lax.axis_index("c")
        sid = lax.axis_index("s")
        wid = sid * NC + cid
        b = wid // 2
        h = wid % 2
        base_row = b * T
        iota = lax.iota(jnp.int32, L)
        filler = jnp.full((L,), base_row, jnp.int32)

        pltpu.sync_copy(dur_hbm, dur_all)

        def init_body(i, _):
            idx_v[pl.ds(i * L, L)] = filler
            return 0
        lax.fori_loop(0, MAXLEN // L, init_body, 0)

        def expand_body(i, carry):
            d = dur_all[b, pl.ds(i * L, L)]
            csum = plsc.cumsum(d) + carry
            start = csum - d
            rows = base_row + i * L + iota
            for rep in range(DMAX):
                pos = start + rep
                m = (rep < d) & (pos < MAXLEN)
                pos_c = jnp.minimum(pos, MAXLEN - 1)
                plsc.store_scatter(idx_v, [pos_c], rows, mask=m)
            return jnp.max(csum)

        mel = lax.fori_loop(0, T // L, expand_body, jnp.int32(0))
        valid = jnp.minimum(mel, MAXLEN)

        zf = jnp.zeros((L,), jnp.float32)

        def chunk_body(c, _):
            cg = h * NCHUNKS + c
            j0 = cg * CHUNK
            pltpu.async_copy(
                x_hbm.at[idx_v.at[pl.ds(j0, CHUNK)]], rows_v, sem).wait()

            r0 = jnp.clip(valid - j0, 0, CHUNK)

            def fix_body(r, _):
                for k in range(D // L):
                    rows_v[r, pl.ds(k * L, L)] = zf
                return 0
            lax.fori_loop(r0, CHUNK, fix_body, 0)

            pltpu.sync_copy(rows_v, out_hbm.at[pl.ds(b * MAXLEN + j0, CHUNK)])
            return 0
        lax.fori_loop(0, NCHUNKS, chunk_body, 0)

        @pl.when(wid == 0)
        def _():
            def mel_body(b2, acc):
                def sum_body(i, s):
                    return s + jnp.sum(dur_all[b2, pl.ds(i * L, L)])
                tot = lax.fori_loop(0, T // L, sum_body, jnp.int32(0))
                return acc + jnp.where(iota == b2, tot, 0)
            mv = lax.fori_loop(0, B, mel_body, jnp.zeros((L,), jnp.int32))
            mel_v[...] = mv
            pltpu.sync_copy(mel_v, mel_hbm)

    return expand


_EXPAND = _build()


@jax.jit
def _run(table, duration):
    return _EXPAND(table, duration)


def kernel(x, duration, max_len):
    table = x.reshape(B * T, D)
    out_flat, mel = _run(table, duration)
    return out_flat.reshape(B, MAXLEN, D), mel


# speedup vs baseline: 50.7866x; 1.0322x over previous
import functools

import jax
import jax.numpy as jnp
from jax import lax
from jax.experimental import pallas as pl
from jax.experimental.pallas import tpu as pltpu
from jax.experimental.pallas import tpu_sc as plsc

B = 16
T = 512
D = 256
MAXLEN = 2048
L = 16
NC = 2
NS = 16
NW = NC * NS
POS_PER_W = B * MAXLEN // NW
CHUNK = 128
NCHUNKS = POS_PER_W // CHUNK
DMAX = 7


def _build():
    mesh = plsc.VectorSubcoreMesh(core_axis_name="c", subcore_axis_name="s")

    @functools.partial(
        pl.kernel,
        mesh=mesh,
        compiler_params=pltpu.CompilerParams(needs_layout_passes=False),
        out_type=(
            jax.ShapeDtypeStruct((B * MAXLEN, D), jnp.float32),
            jax.ShapeDtypeStruct((B,), jnp.int32),
        ),
        scratch_types=[
            pltpu.VMEM((B, T), jnp.int32),
            pltpu.VMEM((MAXLEN,), jnp.int32),
            pltpu.VMEM((CHUNK, D), jnp.float32),
            pltpu.VMEM((CHUNK, D), jnp.float32),
            pltpu.VMEM((CHUNK, D), jnp.float32),
            pltpu.VMEM((L,), jnp.int32),
            pltpu.SemaphoreType.DMA,
            pltpu.SemaphoreType.DMA,
            pltpu.SemaphoreType.DMA,
            pltpu.SemaphoreType.DMA,
            pltpu.SemaphoreType.DMA,
            pltpu.SemaphoreType.DMA,
        ],
    )
    def expand(x_hbm, dur_hbm, out_hbm, mel_hbm, dur_all, idx_v, rows_v0,
               rows_v1, rows_v2, mel_v, gs0, gs1, gs2, ws0, ws1, ws2):
        cid = lax.axis_index("c")
        sid = lax.axis_index("s")
        wid = sid * NC + cid
        b = wid // 2
        h = wid % 2
        base_row = b * T
        iota = lax.iota(jnp.int32, L)
        filler = jnp.full((L,), base_row, jnp.int32)

        pltpu.sync_copy(dur_hbm, dur_all)

        def init_body(i, _):
            idx_v[pl.ds(i * L, L)] = filler
            return 0
        lax.fori_loop(0, MAXLEN // L, init_body, 0)

        def expand_body(i, carry):
            d = dur_all[b, pl.ds(i * L, L)]
            csum = plsc.cumsum(d) + carry
            start = csum - d
            rows = base_row + i * L + iota
            for rep in range(DMAX):
                pos = start + rep
                m = (rep < d) & (pos < MAXLEN)
                pos_c = jnp.minimum(pos, MAXLEN - 1)
                plsc.store_scatter(idx_v, [pos_c], rows, mask=m)
            return jnp.max(csum)

        mel = lax.fori_loop(0, T // L, expand_body, jnp.int32(0))
        valid = jnp.minimum(mel, MAXLEN)

        zf = jnp.zeros((L,), jnp.float32)
        bufs = (rows_v0, rows_v1, rows_v2)
        gsems = (gs0, gs1, gs2)
        wsems = (ws0, ws1, ws2)

        def fixup_and_write(c, buf, wsem):
            j0 = (h * NCHUNKS + c) * CHUNK
            r0 = jnp.clip(valid - j0, 0, CHUNK)

            def fix_body(r, _):
                for k in range(D // L):
                    buf[r, pl.ds(k * L, L)] = zf
                return 0
            lax.fori_loop(r0, CHUNK, fix_body, 0)
            return pltpu.async_copy(
                buf, out_hbm.at[pl.ds(b * MAXLEN + j0, CHUNK)], wsem)

        def gather(c, buf, gsem):
            j0 = (h * NCHUNKS + c) * CHUNK
            return pltpu.async_copy(
                x_hbm.at[idx_v.at[pl.ds(j0, CHUNK)]], buf, gsem)

        g = [None] * NCHUNKS
        w = [None] * NCHUNKS
        for c in range(NCHUNKS):
            bi = c % 3
            if c >= 3:
                w[c - 3].wait()
            g[c] = gather(c, bufs[bi], gsems[bi])
            if c >= 1:
                pi = (c - 1) % 3
                g[c - 1].wait()
                w[c - 1] = fixup_and_write(c - 1, bufs[pi], wsems[pi])
        g[NCHUNKS - 1].wait()
        w[NCHUNKS - 1] = fixup_and_write(
            NCHUNKS - 1, bufs[(NCHUNKS - 1) % 3], wsems[(NCHUNKS - 1) % 3])
        for c in range(NCHUNKS - 3, NCHUNKS):
            w[c].wait()

        @pl.when(wid == 0)
        def _():
            def mel_body(b2, acc):
                def sum_body(i, s):
                    return s + jnp.sum(dur_all[b2, pl.ds(i * L, L)])
                tot = lax.fori_loop(0, T // L, sum_body, jnp.int32(0))
                return acc + jnp.where(iota == b2, tot, 0)
            mv = lax.fori_loop(0, B, mel_body, jnp.zeros((L,), jnp.int32))
            mel_v[...] = mv
            pltpu.sync_copy(mel_v, mel_hbm)

    return expand


_EXPAND = _build()


@jax.jit
def _run(table, duration):
    return _EXPAND(table, duration)


def kernel(x, duration, max_len):
    table = x.reshape(B * T, D)
    out_flat, mel = _run(table, duration)
    return out_flat.reshape(B, MAXLEN, D), mel


# speedup vs baseline: 53.8672x; 1.0607x over previous
import functools

import jax
import jax.numpy as jnp
from jax import lax
from jax.experimental import pallas as pl
from jax.experimental.pallas import tpu as pltpu
from jax.experimental.pallas import tpu_sc as plsc

B = 16
T = 512
D = 256
MAXLEN = 2048
L = 16
NC = 2
NS = 16
NW = NC * NS
POS_PER_W = B * MAXLEN // NW
CHUNK = 128
NCHUNKS = POS_PER_W // CHUNK
DMAX = 7


def _build():
    mesh = plsc.VectorSubcoreMesh(core_axis_name="c", subcore_axis_name="s")

    @functools.partial(
        pl.kernel,
        mesh=mesh,
        compiler_params=pltpu.CompilerParams(needs_layout_passes=False),
        out_type=(
            jax.ShapeDtypeStruct((B * MAXLEN, D), jnp.float32),
            jax.ShapeDtypeStruct((B,), jnp.int32),
        ),
        scratch_types=[
            pltpu.VMEM((B, T), jnp.int32),
            pltpu.VMEM((MAXLEN,), jnp.int32),
            pltpu.VMEM((CHUNK, D), jnp.float32),
            pltpu.VMEM((CHUNK, D), jnp.float32),
            pltpu.VMEM((CHUNK, D), jnp.float32),
            pltpu.VMEM((L,), jnp.int32),
            pltpu.SemaphoreType.DMA,
            pltpu.SemaphoreType.DMA,
            pltpu.SemaphoreType.DMA,
            pltpu.SemaphoreType.DMA,
            pltpu.SemaphoreType.DMA,
            pltpu.SemaphoreType.DMA,
        ],
    )
    def expand(x_hbm, dur_hbm, out_hbm, mel_hbm, dur_all, idx_v, rows_v0,
               rows_v1, rows_v2, mel_v, gs0, gs1, gs2, ws0, ws1, ws2):
        cid = lax.axis_index("c")
        sid = lax.axis_index("s")
        wid = sid * NC + cid
        b = wid // 2
        h = wid % 2
        base_row = b * T
        iota = lax.iota(jnp.int32, L)
        filler = jnp.full((L,), base_row, jnp.int32)

        pltpu.sync_copy(dur_hbm, dur_all)

        def init_body(i, _):
            idx_v[pl.ds(i * L, L)] = filler
            return 0
        lax.fori_loop(0, MAXLEN // L, init_body, 0)

        def expand_body(i, carry):
            d = dur_all[b, pl.ds(i * L, L)]
            csum = plsc.cumsum(d) + carry
            start = csum - d
            rows = base_row + i * L + iota
            for rep in range(DMAX):
                pos = start + rep
                m = (rep < d) & (pos < MAXLEN)
                pos_c = jnp.minimum(pos, MAXLEN - 1)
                plsc.store_scatter(idx_v, [pos_c], rows, mask=m)
            return jnp.max(csum)

        mel = lax.fori_loop(0, T // L, expand_body, jnp.int32(0))
        valid = jnp.minimum(mel, MAXLEN)

        zf = jnp.zeros((L,), jnp.float32)
        bufs = (rows_v0, rows_v1, rows_v2)
        gsems = (gs0, gs1, gs2)
        wsems = (ws0, ws1, ws2)

        def fixup_and_write(c, buf, wsem):
            j0 = (h + 2 * c) * CHUNK
            r0 = jnp.clip(valid - j0, 0, CHUNK)

            def fix_body(r, _):
                for k in range(D // L):
                    buf[r, pl.ds(k * L, L)] = zf
                return 0
            lax.fori_loop(r0, CHUNK, fix_body, 0)
            return pltpu.async_copy(
                buf, out_hbm.at[pl.ds(b * MAXLEN + j0, CHUNK)], wsem)

        def gather(c, buf, gsem):
            j0 = (h + 2 * c) * CHUNK
            return pltpu.async_copy(
                x_hbm.at[idx_v.at[pl.ds(j0, CHUNK)]], buf, gsem)

        g = [None] * NCHUNKS
        w = [None] * NCHUNKS
        for c in range(NCHUNKS):
            bi = c % 3
            if c >= 3:
                w[c - 3].wait()
            g[c] = gather(c, bufs[bi], gsems[bi])
            if c >= 1:
                pi = (c - 1) % 3
                g[c - 1].wait()
                w[c - 1] = fixup_and_write(c - 1, bufs[pi], wsems[pi])
        g[NCHUNKS - 1].wait()
        w[NCHUNKS - 1] = fixup_and_write(
            NCHUNKS - 1, bufs[(NCHUNKS - 1) % 3], wsems[(NCHUNKS - 1) % 3])
        for c in range(NCHUNKS - 3, NCHUNKS):
            w[c].wait()

        @pl.when(wid == 0)
        def _():
            def mel_body(b2, acc):
                def sum_body(i, s):
                    return s + jnp.sum(dur_all[b2, pl.ds(i * L, L)])
                tot = lax.fori_loop(0, T // L, sum_body, jnp.int32(0))
                return acc + jnp.where(iota == b2, tot, 0)
            mv = lax.fori_loop(0, B, mel_body, jnp.zeros((L,), jnp.int32))
            mel_v[...] = mv
            pltpu.sync_copy(mel_v, mel_hbm)

    return expand


_EXPAND = _build()


@jax.jit
def _run(table, duration):
    return _EXPAND(table, duration)


def kernel(x, duration, max_len):
    table = x.reshape(B * T, D)
    out_flat, mel = _run(table, duration)
    return out_flat.reshape(B, MAXLEN, D), mel


# speedup vs baseline: 61.7932x; 1.1471x over previous
import functools

import jax
import jax.numpy as jnp
from jax import lax
from jax.experimental import pallas as pl
from jax.experimental.pallas import tpu as pltpu
from jax.experimental.pallas import tpu_sc as plsc

B = 16
T = 512
D = 256
MAXLEN = 2048
L = 16
NC = 2
NS = 16
NW = NC * NS
CHUNK = 128
NCHUNKS = MAXLEN // CHUNK // 2
DMAX = 7
NBUF = 3


def _build():
    mesh = plsc.VectorSubcoreMesh(core_axis_name="c", subcore_axis_name="s")

    @functools.partial(
        pl.kernel,
        mesh=mesh,
        compiler_params=pltpu.CompilerParams(needs_layout_passes=False),
        out_type=(
            jax.ShapeDtypeStruct((B * MAXLEN, D), jnp.float32),
            jax.ShapeDtypeStruct((B,), jnp.int32),
        ),
        scratch_types=[
            pltpu.VMEM((B, T), jnp.int32),
            pltpu.VMEM((MAXLEN,), jnp.int32),
            pltpu.VMEM((CHUNK, D), jnp.float32),
            pltpu.VMEM((CHUNK, D), jnp.float32),
            pltpu.VMEM((CHUNK, D), jnp.float32),
            pltpu.VMEM((L,), jnp.int32),
            pltpu.VMEM_SHARED((CHUNK, D), jnp.float32),
            pltpu.SemaphoreType.DMA,
            pltpu.SemaphoreType.DMA,
            pltpu.SemaphoreType.DMA,
            pltpu.SemaphoreType.DMA,
            pltpu.SemaphoreType.DMA,
            pltpu.SemaphoreType.DMA,
        ],
    )
    def expand(x_hbm, dur_hbm, out_hbm, mel_hbm, dur_all, idx_v, rows_v0,
               rows_v1, rows_v2, mel_v, zsp,
               gs0, gs1, gs2, ws0, ws1, ws2):
        cid = lax.axis_index("c")
        sid = lax.axis_index("s")
        wid = sid * NC + cid
        b = wid // 2
        h = wid % 2
        base_row = b * T
        iota = lax.iota(jnp.int32, L)
        filler = jnp.full((L,), base_row, jnp.int32)
        zf = jnp.zeros((L,), jnp.float32)
        bufs = (rows_v0, rows_v1, rows_v2)
        gsems = (gs0, gs1, gs2)
        wsems = (ws0, ws1, ws2)

        pltpu.sync_copy(dur_hbm, dur_all)

        def expand_body(i, carry):
            d = dur_all[b, pl.ds(i * L, L)]
            csum = plsc.cumsum(d) + carry
            start = csum - d
            rows = base_row + i * L + iota
            for rep in range(DMAX):
                pos = start + rep
                m = (rep < d) & (pos < MAXLEN)
                pos_c = jnp.minimum(pos, MAXLEN - 1)
                plsc.store_scatter(idx_v, [pos_c], rows, mask=m)
            return jnp.max(csum)

        mel = lax.fori_loop(0, T // L, expand_body, jnp.int32(0))
        valid = jnp.minimum(mel, MAXLEN)

        bc_base = jnp.minimum(valid // CHUNK, MAXLEN // CHUNK - 1) * CHUNK
        for v in range(CHUNK // L):
            pos = bc_base + v * L + iota
            plsc.store_scatter(idx_v, [pos], filler, mask=pos >= valid)

        @pl.when(sid == 0)
        def _():
            def zb(r, _):
                for k in range(D // L):
                    rows_v0[r, pl.ds(k * L, L)] = zf
                return 0
            lax.fori_loop(0, CHUNK, zb, 0)
            pltpu.sync_copy(rows_v0, zsp)

        plsc.subcore_barrier()

        @pl.when(wid == 0)
        def _():
            def mel_body(b2, acc):
                def sum_body(i, s):
                    return s + jnp.sum(dur_all[b2, pl.ds(i * L, L)])
                tot = lax.fori_loop(0, T // L, sum_body, jnp.int32(0))
                return acc + jnp.where(iota == b2, tot, 0)
            mv = lax.fori_loop(0, B, mel_body, jnp.zeros((L,), jnp.int32))
            mel_v[...] = mv
            pltpu.sync_copy(mel_v, mel_hbm)

        def j0_of(c):
            return (h + 2 * c) * CHUNK

        def gsrc(c):
            return x_hbm.at[idx_v.at[pl.ds(j0_of(c), CHUNK)]]

        def odst(c):
            return out_hbm.at[pl.ds(b * MAXLEN + j0_of(c), CHUNK)]

        preds = [j0_of(c) < valid for c in range(NCHUNKS)]

        def finish_chunk(c):
            bi = c % NBUF

            @pl.when(preds[c])
            def _():
                pltpu.make_async_copy(gsrc(c), bufs[bi], gsems[bi]).wait()
                r0 = jnp.clip(valid - j0_of(c), 0, CHUNK)

                def fix(r, _):
                    for k in range(D // L):
                        bufs[bi][r, pl.ds(k * L, L)] = zf
                    return 0
                lax.fori_loop(r0, CHUNK, fix, 0)
                pltpu.async_copy(bufs[bi], odst(c), wsems[bi])

            @pl.when(jnp.logical_not(preds[c]))
            def _():
                pltpu.sync_copy(zsp, odst(c))

        for c in range(NCHUNKS):
            bi = c % NBUF
            if c >= NBUF:
                @pl.when(preds[c - NBUF])
                def _(c=c, bi=bi):
                    pltpu.make_async_copy(
                        bufs[bi], odst(c - NBUF), wsems[bi]).wait()

            @pl.when(preds[c])
            def _(c=c, bi=bi):
                pltpu.async_copy(gsrc(c), bufs[bi], gsems[bi])

            if c >= 1:
                finish_chunk(c - 1)
        finish_chunk(NCHUNKS - 1)

        for c in range(NCHUNKS - NBUF, NCHUNKS):
            @pl.when(preds[c])
            def _(c=c):
                pltpu.make_async_copy(
                    bufs[c % NBUF], odst(c), wsems[c % NBUF]).wait()

    return expand


_EXPAND = _build()


@jax.jit
def _run(table, duration):
    return _EXPAND(table, duration)


def kernel(x, duration, max_len):
    table = x.reshape(B * T, D)
    out_flat, mel = _run(table, duration)
    return out_flat.reshape(B, MAXLEN, D), mel


# speedup vs baseline: 64.6921x; 1.0469x over previous
import functools

import jax
import jax.numpy as jnp
from jax import lax
from jax.experimental import pallas as pl
from jax.experimental.pallas import tpu as pltpu
from jax.experimental.pallas import tpu_sc as plsc

B = 16
T = 512
D = 256
MAXLEN = 2048
L = 16
NC = 2
NS = 16
NW = NC * NS
CHUNK = 128
NCHUNKS = MAXLEN // CHUNK // 2
DMAX = 7
NBUF = 3
PRE = 2


def _build():
    mesh = plsc.VectorSubcoreMesh(core_axis_name="c", subcore_axis_name="s")

    @functools.partial(
        pl.kernel,
        mesh=mesh,
        compiler_params=pltpu.CompilerParams(needs_layout_passes=False),
        out_type=(
            jax.ShapeDtypeStruct((B * MAXLEN, D), jnp.float32),
            jax.ShapeDtypeStruct((B,), jnp.int32),
        ),
        scratch_types=[
            pltpu.VMEM((B, T), jnp.int32),
            pltpu.VMEM((MAXLEN,), jnp.int32),
            pltpu.VMEM((CHUNK, D), jnp.float32),
            pltpu.VMEM((CHUNK, D), jnp.float32),
            pltpu.VMEM((CHUNK, D), jnp.float32),
            pltpu.VMEM((L,), jnp.int32),
            pltpu.VMEM_SHARED((CHUNK, D), jnp.float32),
            pltpu.SemaphoreType.DMA,
            pltpu.SemaphoreType.DMA,
            pltpu.SemaphoreType.DMA,
            pltpu.SemaphoreType.DMA,
            pltpu.SemaphoreType.DMA,
            pltpu.SemaphoreType.DMA,
            pltpu.SemaphoreType.DMA,
        ],
    )
    def expand(x_hbm, dur_hbm, out_hbm, mel_hbm, dur_all, idx_v, rows_v0,
               rows_v1, rows_v2, mel_v, zsp,
               gs0, gs1, gs2, ws0, ws1, ws2, ts):
        cid = lax.axis_index("c")
        sid = lax.axis_index("s")
        wid = sid * NC + cid
        b = wid // 2
        h = wid % 2
        base_row = b * T
        iota = lax.iota(jnp.int32, L)
        filler = jnp.full((L,), base_row, jnp.int32)
        zf = jnp.zeros((L,), jnp.float32)
        bufs = (rows_v0, rows_v1, rows_v2)
        gsems = (gs0, gs1, gs2)
        wsems = (ws0, ws1, ws2)

        pltpu.sync_copy(dur_hbm, dur_all)

        def expand_body(i, carry):
            d = dur_all[b, pl.ds(i * L, L)]
            csum = plsc.cumsum(d) + carry
            start = csum - d
            rows = base_row + i * L + iota
            for rep in range(DMAX):
                pos = start + rep
                m = (rep < d) & (pos < MAXLEN)
                pos_c = jnp.minimum(pos, MAXLEN - 1)
                plsc.store_scatter(idx_v, [pos_c], rows, mask=m)
            return jnp.max(csum)

        mel = lax.fori_loop(0, T // L, expand_body, jnp.int32(0))
        valid = jnp.minimum(mel, MAXLEN)

        bc_base = jnp.minimum(valid // CHUNK, MAXLEN // CHUNK - 1) * CHUNK
        for v in range(CHUNK // L):
            pos = bc_base + v * L + iota
            plsc.store_scatter(idx_v, [pos], filler, mask=pos >= valid)

        def j0_of(c):
            return (h + 2 * c) * CHUNK

        def gsrc(c):
            return x_hbm.at[idx_v.at[pl.ds(j0_of(c), CHUNK)]]

        def odst(c):
            return out_hbm.at[pl.ds(b * MAXLEN + j0_of(c), CHUNK)]

        preds = [j0_of(c) < valid for c in range(NCHUNKS)]

        for c in range(PRE):
            @pl.when(preds[c])
            def _(c=c):
                pltpu.async_copy(gsrc(c), bufs[c % NBUF], gsems[c % NBUF])

        ZW = 4
        ZROWS = CHUNK // ZW

        @pl.when(sid >= NS - ZW)
        def _():
            def zb(r, _):
                for k in range(D // L):
                    rows_v2[r, pl.ds(k * L, L)] = zf
                return 0
            lax.fori_loop(0, ZROWS, zb, 0)
            pltpu.sync_copy(rows_v2.at[pl.ds(0, ZROWS)],
                            zsp.at[pl.ds((sid - (NS - ZW)) * ZROWS, ZROWS)])

        plsc.subcore_barrier()

        def finish_chunk(c):
            bi = c % NBUF

            @pl.when(preds[c])
            def _():
                pltpu.make_async_copy(gsrc(c), bufs[bi], gsems[bi]).wait()
                r0 = jnp.clip(valid - j0_of(c), 0, CHUNK)

                def fix(r, _):
                    for k in range(D // L):
                        bufs[bi][r, pl.ds(k * L, L)] = zf
                    return 0
                lax.fori_loop(r0, CHUNK, fix, 0)
                pltpu.async_copy(bufs[bi], odst(c), wsems[bi])

            @pl.when(jnp.logical_not(preds[c]))
            def _():
                pltpu.async_copy(zsp, odst(c), ts)

        for c in range(NCHUNKS):
            bi = c % NBUF
            if c >= NBUF:
                @pl.when(preds[c - NBUF])
                def _(c=c, bi=bi):
                    pltpu.make_async_copy(
                        bufs[bi], odst(c - NBUF), wsems[bi]).wait()

            if c >= PRE:
                @pl.when(preds[c])
                def _(c=c, bi=bi):
                    pltpu.async_copy(gsrc(c), bufs[bi], gsems[bi])

            if c >= 1:
                finish_chunk(c - 1)
        finish_chunk(NCHUNKS - 1)

        @pl.when(wid == 0)
        def _():
            def mel_body(b2, acc):
                def sum_body(i, s):
                    return s + dur_all[b2, pl.ds(i * L, L)]
                sv = lax.fori_loop(0, T // L, sum_body,
                                   jnp.zeros((L,), jnp.int32))
                return acc + jnp.where(iota == b2, jnp.sum(sv), 0)
            mv = lax.fori_loop(0, B, mel_body, jnp.zeros((L,), jnp.int32))
            mel_v[...] = mv
            pltpu.sync_copy(mel_v, mel_hbm)

        for c in range(NCHUNKS - NBUF, NCHUNKS):
            @pl.when(preds[c])
            def _(c=c):
                pltpu.make_async_copy(
                    bufs[c % NBUF], odst(c), wsems[c % NBUF]).wait()

        for c in range(NCHUNKS):
            @pl.when(jnp.logical_not(preds[c]))
            def _(c=c):
                pltpu.make_async_copy(zsp, odst(c), ts).wait()

    return expand


_EXPAND = _build()


@jax.jit
def _run(table, duration):
    return _EXPAND(table, duration)


def kernel(x, duration, max_len):
    table = x.reshape(B * T, D)
    out_flat, mel = _run(table, duration)
    return out_flat.reshape(B, MAXLEN, D), mel
